# Initial kernel scaffold; baseline (speedup 1.0000x reference)
import jax
import jax.numpy as jnp
from jax.experimental import pallas as pl


def kernel(x, edge_index, edge_weight, xz_W0, xz_W1, xz_b, hz_W0, hz_W1, hz_b, xr_W0, xr_W1, xr_b, hr_W0, hr_W1, hr_b, xh_W0, xh_W1, xh_b, hh_W0, hh_W1, hh_b, lin_W, lin_b):
    raise NotImplementedError("write your pallas kernel here")


# speedup vs baseline: 1.5936x; 1.5936x over previous
import functools

import jax
import jax.numpy as jnp
from jax.experimental import pallas as pl
from jax.experimental.pallas import tpu as pltpu

N = 10000
F = 128
BLK = 1000


def _dense_body(x_ref, t_ref, wz0_ref, wz1_ref, wh0_ref, wh1_ref,
                bz_ref, bh_ref, lw_ref, lb_ref, out_ref):
    xb = x_ref[...]
    tb = t_ref[...]
    z = jax.nn.sigmoid(xb @ wz0_ref[...] + tb @ wz1_ref[...] + bz_ref[...])
    ht = jnp.tanh(xb @ wh0_ref[...] + tb @ wh1_ref[...] + bh_ref[...])
    h = jax.nn.relu((1.0 - z) * ht)
    out_ref[...] = h @ lw_ref[...] + lb_ref[...]


def _dense_block(x, tx1, wz0, wz1, wh0, wh1, bz, bh, lw, lb):
    row_spec = pl.BlockSpec((BLK, F), lambda i: (i, 0))
    full = lambda shape: pl.BlockSpec(shape, lambda i: (0, 0))
    return pl.pallas_call(
        _dense_body,
        grid=(N // BLK,),
        in_specs=[row_spec, row_spec,
                  full((F, F)), full((F, F)), full((F, F)), full((F, F)),
                  full((1, F)), full((1, F)), full((F, 1)), full((1, 1))],
        out_specs=pl.BlockSpec((BLK, 1), lambda i: (i, 0)),
        out_shape=jax.ShapeDtypeStruct((N, 1), jnp.float32),
    )(x, tx1, wz0, wz1, wh0, wh1, bz, bh, lw, lb)


def kernel(x, edge_index, edge_weight,
           xz_W0, xz_W1, xz_b, hz_W0, hz_W1, hz_b,
           xr_W0, xr_W1, xr_b, hr_W0, hr_W1, hr_b,
           xh_W0, xh_W1, xh_b, hh_W0, hh_W1, hh_b,
           lin_W, lin_b):
    src = edge_index[0]
    dst = edge_index[1]
    deg = jax.ops.segment_sum(edge_weight, src, num_segments=N)
    dis = jnp.where(deg > 0, jax.lax.rsqrt(jnp.where(deg > 0, deg, 1.0)), 0.0)
    norm = -edge_weight * dis[src] * dis[dst]
    tx1 = jax.ops.segment_sum(norm[:, None] * x[src], dst, num_segments=N)
    bz = (xz_b + hz_b).reshape(1, F)
    bh = (xh_b + hh_b).reshape(1, F)
    return _dense_block(x, tx1, xz_W0, xz_W1, xh_W0, xh_W1,
                        bz, bh, lin_W, lin_b.reshape(1, 1))


# speedup vs baseline: 19.7828x; 12.4138x over previous
import functools

import jax
import jax.numpy as jnp
from jax import lax
from jax.experimental import pallas as pl
from jax.experimental.pallas import tpu as pltpu
from jax.experimental.pallas import tpu_sc as plsc

N = 10000
F = 128
FH = F // 2
E = 320000
BLK = 1000

NC = 2
NS = 16
NW = NC * NS
EB = E // NW
RB = 79
EBP = RB * 128
DEG_N = 10240
ORPT = DEG_N // NS


def _spmm_body(src_hbm, dst_hbm, w_hbm, xflat_hbm,
               out_tx, out_degp, out_dis,
               src2d, dst2d, w2d, norm2d, degloc, degtmp, zrow, rows,
               tx1_sh, gsem):
    c = lax.axis_index("c")
    s = lax.axis_index("s")

    zero16 = jnp.zeros((16,), jnp.float32)

    def zr(i, _):
        degloc[pl.ds(16 * i, 16)] = zero16
        return 0
    lax.fori_loop(0, DEG_N // 16, zr, 0)

    def zw(i, _):
        for k in range(4):
            zrow[i, pl.ds(16 * k, 16)] = zero16
        return 0
    lax.fori_loop(0, 160, zw, 0)

    for k in range(4):
        pltpu.sync_copy(zrow, tx1_sh.at[pl.ds(ORPT * s + 160 * k, 160)])

    for half in range(2):
        b = 2 * s + half
        pltpu.sync_copy(src_hbm.at[b], src2d)
        pltpu.sync_copy(w_hbm.at[b], w2d)

        def ar(r, _):
            for j in range(8):
                idx = src2d[r, pl.ds(16 * j, 16)]
                wv = w2d[r, pl.ds(16 * j, 16)]
                plsc.addupdate_scatter(degloc, [idx], wv)
            return 0
        lax.fori_loop(0, RB, ar, 0)

    pltpu.sync_copy(degloc, out_degp.at[c, s])
    plsc.subcore_barrier()
    for t in range(NS):
        pltpu.sync_copy(out_degp.at[c, t, pl.ds(ORPT * s, ORPT)],
                        degtmp.at[t])

    def mr(i, _):
        acc = degtmp[0, pl.ds(16 * i, 16)]
        for t in range(1, NS):
            acc = acc + degtmp[t, pl.ds(16 * i, 16)]
        iv = plsc.bitcast(acc, jnp.int32)
        y = plsc.bitcast(0x5F3759DF - lax.shift_right_logical(iv, 1),
                         jnp.float32)
        for _it in range(3):
            y = y * (1.5 - ((0.5 * acc) * y) * y)
        degtmp[0, pl.ds(16 * i, 16)] = jnp.where(acc > 0.0, y, 0.0)
        return 0
    lax.fori_loop(0, ORPT // 16, mr, 0)

    pltpu.sync_copy(degtmp.at[0], out_dis.at[c, pl.ds(ORPT * s, ORPT)])
    plsc.subcore_barrier()
    pltpu.sync_copy(out_dis.at[c], degloc)

    xoff = c * N
    for half in range(2):
        b = 2 * s + half
        pltpu.sync_copy(src_hbm.at[b], src2d)
        pltpu.sync_copy(dst_hbm.at[b], dst2d)
        pltpu.sync_copy(w_hbm.at[b], w2d)

        def nr(r, _):
            for j in range(8):
                si = src2d[r, pl.ds(16 * j, 16)]
                di = dst2d[r, pl.ds(16 * j, 16)]
                wv = w2d[r, pl.ds(16 * j, 16)]
                dsv = plsc.load_gather(degloc, [si])
                ddv = plsc.load_gather(degloc, [di])
                norm2d[r, pl.ds(16 * j, 16)] = (-wv) * dsv * ddv
                src2d[r, pl.ds(16 * j, 16)] = si + xoff
            return 0
        lax.fori_loop(0, RB, nr, 0)

        def cr(r, _):
            pltpu.async_copy(xflat_hbm.at[src2d.at[r]], rows, gsem).wait()

            def er(g, _):
                nv = norm2d[r, pl.ds(16 * g, 16)]
                for e in range(16):
                    nb = jnp.full((16,), nv[e], jnp.float32)
                    for k in range(4):
                        rows[16 * g + e, pl.ds(16 * k, 16)] = (
                            rows[16 * g + e, pl.ds(16 * k, 16)] * nb)
                return 0
            lax.fori_loop(0, 8, er, 0)

            pltpu.sync_copy(rows, tx1_sh.at[dst2d.at[r]], add=True)
            return 0
        lax.fori_loop(0, RB, cr, 0)

    plsc.subcore_barrier()

    pltpu.sync_copy(tx1_sh.at[pl.ds(ORPT * s, ORPT)],
                    out_tx.at[c, pl.ds(ORPT * s, ORPT)])


@jax.jit
def _spmm_sc(src_p, dst_p, w_p, xflat):
    mesh = plsc.VectorSubcoreMesh(core_axis_name="c", subcore_axis_name="s",
                                  num_cores=NC, num_subcores=NS)
    return pl.kernel(
        _spmm_body,
        out_type=(
            jax.ShapeDtypeStruct((NC, DEG_N, FH), jnp.float32),
            jax.ShapeDtypeStruct((NC, NS, DEG_N), jnp.float32),
            jax.ShapeDtypeStruct((NC, DEG_N), jnp.float32),
        ),
        mesh=mesh,
        compiler_params=pltpu.CompilerParams(needs_layout_passes=False,
                                             use_tc_tiling_on_sc=False),
        scratch_types=[
            pltpu.VMEM((RB, 128), jnp.int32),
            pltpu.VMEM((RB, 128), jnp.int32),
            pltpu.VMEM((RB, 128), jnp.float32),
            pltpu.VMEM((RB, 128), jnp.float32),
            pltpu.VMEM((DEG_N,), jnp.float32),
            pltpu.VMEM((NS, ORPT), jnp.float32),
            pltpu.VMEM((160, FH), jnp.float32),
            pltpu.VMEM((128, FH), jnp.float32),
            pltpu.VMEM_SHARED((DEG_N, FH), jnp.float32),
            pltpu.SemaphoreType.DMA,
        ],
    )(src_p, dst_p, w_p, xflat)


def _dense_body(x_ref, t0_ref, t1_ref, wz0_ref, wz1a_ref, wz1b_ref,
                wh0_ref, wh1a_ref, wh1b_ref,
                bz_ref, bh_ref, lw_ref, lb_ref, out_ref):
    xb = x_ref[...]
    t0 = t0_ref[0]
    t1 = t1_ref[0]
    z = jax.nn.sigmoid(xb @ wz0_ref[...] + t0 @ wz1a_ref[...]
                       + t1 @ wz1b_ref[...] + bz_ref[...])
    ht = jnp.tanh(xb @ wh0_ref[...] + t0 @ wh1a_ref[...]
                  + t1 @ wh1b_ref[...] + bh_ref[...])
    h = jax.nn.relu((1.0 - z) * ht)
    out_ref[...] = h @ lw_ref[...] + lb_ref[...]


def _dense_block(x, parts, wz0, wz1, wh0, wh1, bz, bh, lw, lb):
    row_spec = pl.BlockSpec((BLK, F), lambda i: (i, 0))
    part0 = pl.BlockSpec((1, BLK, FH), lambda i: (0, i, 0))
    part1 = pl.BlockSpec((1, BLK, FH), lambda i: (1, i, 0))
    full = lambda shape: pl.BlockSpec(shape, lambda i: (0, 0))
    return pl.pallas_call(
        _dense_body,
        grid=(N // BLK,),
        in_specs=[row_spec, part0, part1,
                  full((F, F)), full((FH, F)), full((FH, F)),
                  full((F, F)), full((FH, F)), full((FH, F)),
                  full((1, F)), full((1, F)), full((F, 1)), full((1, 1))],
        out_specs=pl.BlockSpec((BLK, 1), lambda i: (i, 0)),
        out_shape=jax.ShapeDtypeStruct((N, 1), jnp.float32),
    )(x, parts, parts, wz0, wz1[:FH], wz1[FH:], wh0, wh1[:FH], wh1[FH:],
      bz, bh, lw, lb)


def kernel(x, edge_index, edge_weight,
           xz_W0, xz_W1, xz_b, hz_W0, hz_W1, hz_b,
           xr_W0, xr_W1, xr_b, hr_W0, hr_W1, hr_b,
           xh_W0, xh_W1, xh_b, hh_W0, hh_W1, hh_b,
           lin_W, lin_b):
    def pad_blocks(a, pad_val):
        a = a.reshape(NW, EB)
        a = jnp.pad(a, ((0, 0), (0, EBP - EB)), constant_values=pad_val)
        return a.reshape(NW, RB, 128)

    src_p = pad_blocks(edge_index[0], 0)
    dst_p = pad_blocks(edge_index[1], 0)
    w_p = pad_blocks(edge_weight, 0.0)

    xflat = jnp.concatenate([x[:, :FH], x[:, FH:]], axis=0)

    parts, _, _ = _spmm_sc(src_p, dst_p, w_p, xflat)

    bz = (xz_b + hz_b).reshape(1, F)
    bh = (xh_b + hh_b).reshape(1, F)
    return _dense_block(x, parts, xz_W0, xz_W1, xh_W0, xh_W1,
                        bz, bh, lin_W, lin_b.reshape(1, 1))


# speedup vs baseline: 27.1749x; 1.3737x over previous
import functools

import jax
import jax.numpy as jnp
from jax import lax
from jax.experimental import pallas as pl
from jax.experimental.pallas import tpu as pltpu
from jax.experimental.pallas import tpu_sc as plsc

N = 10000
F = 128
FH = F // 2
E = 320000
BLK = 1000

NC = 2
NS = 16
NW = NC * NS
EB = E // NW
RB = 79
EBP = RB * 128
DEG_N = 10240
ORPT = DEG_N // NS


def _spmm_body(src_hbm, dst_hbm, w_hbm, xflat_hbm,
               out_tx, out_degp, out_dis,
               src2d, dst2d, w2d, norm2d, degloc, degtmp, zrow, rows,
               tx1_sh, gsem, ssem):
    c = lax.axis_index("c")
    s = lax.axis_index("s")

    zero16 = jnp.zeros((16,), jnp.float32)

    def zr(i, _):
        degloc[pl.ds(16 * i, 16)] = zero16
        return 0
    lax.fori_loop(0, DEG_N // 16, zr, 0)

    def zw(i, _):
        for k in range(4):
            zrow[i, pl.ds(16 * k, 16)] = zero16
        return 0
    lax.fori_loop(0, 160, zw, 0)

    for k in range(4):
        pltpu.sync_copy(zrow, tx1_sh.at[pl.ds(ORPT * s + 160 * k, 160)])

    for half in range(2):
        b = 2 * s + half
        pltpu.sync_copy(src_hbm.at[b], src2d)
        pltpu.sync_copy(w_hbm.at[b], w2d)

        def ar(r, _):
            for j in range(8):
                idx = src2d[r, pl.ds(16 * j, 16)]
                wv = w2d[r, pl.ds(16 * j, 16)]
                plsc.addupdate_scatter(degloc, [idx], wv)
            return 0
        lax.fori_loop(0, RB, ar, 0)

    pltpu.sync_copy(degloc, out_degp.at[c, s])
    plsc.subcore_barrier()
    for t in range(NS):
        pltpu.sync_copy(out_degp.at[c, t, pl.ds(ORPT * s, ORPT)],
                        degtmp.at[t])

    def mr(i, _):
        acc = degtmp[0, pl.ds(16 * i, 16)]
        for t in range(1, NS):
            acc = acc + degtmp[t, pl.ds(16 * i, 16)]
        iv = plsc.bitcast(acc, jnp.int32)
        y = plsc.bitcast(0x5F3759DF - lax.shift_right_logical(iv, 1),
                         jnp.float32)
        for _it in range(3):
            y = y * (1.5 - ((0.5 * acc) * y) * y)
        degtmp[0, pl.ds(16 * i, 16)] = jnp.where(acc > 0.0, y, 0.0)
        return 0
    lax.fori_loop(0, ORPT // 16, mr, 0)

    pltpu.sync_copy(degtmp.at[0], out_dis.at[c, pl.ds(ORPT * s, ORPT)])
    plsc.subcore_barrier()
    pltpu.sync_copy(out_dis.at[c], degloc)

    xoff = c * N
    for half in range(2):
        b = 2 * s + half
        pltpu.sync_copy(src_hbm.at[b], src2d)
        pltpu.sync_copy(dst_hbm.at[b], dst2d)
        pltpu.sync_copy(w_hbm.at[b], w2d)

        def nr(r, _):
            for j in range(8):
                si = src2d[r, pl.ds(16 * j, 16)]
                di = dst2d[r, pl.ds(16 * j, 16)]
                wv = w2d[r, pl.ds(16 * j, 16)]
                dsv = plsc.load_gather(degloc, [si])
                ddv = plsc.load_gather(degloc, [di])
                norm2d[r, pl.ds(16 * j, 16)] = (-wv) * dsv * ddv
                src2d[r, pl.ds(16 * j, 16)] = si + xoff
            return 0
        lax.fori_loop(0, RB, nr, 0)

        def cr(r, _):
            cur = lax.rem(r, 2)
            nxt = 1 - cur

            pltpu.make_async_copy(xflat_hbm.at[src2d.at[r]],
                                  rows.at[cur], gsem).wait()

            @pl.when(r >= 1)
            def _wait_prev_scatter():
                pltpu.make_async_copy(
                    rows.at[nxt], tx1_sh.at[dst2d.at[r - 1]], ssem).wait()

            @pl.when(r + 1 < RB)
            def _prefetch_next():
                pltpu.async_copy(xflat_hbm.at[src2d.at[r + 1]],
                                 rows.at[nxt], gsem)

            def er(g, _):
                nv = norm2d[r, pl.ds(16 * g, 16)]
                for e in range(16):
                    nb = jnp.full((16,), nv[e], jnp.float32)
                    for k in range(4):
                        rows[cur, 16 * g + e, pl.ds(16 * k, 16)] = (
                            rows[cur, 16 * g + e, pl.ds(16 * k, 16)] * nb)
                return 0
            lax.fori_loop(0, 8, er, 0)

            pltpu.async_copy(rows.at[cur], tx1_sh.at[dst2d.at[r]], ssem,
                             add=True)
            return 0

        pltpu.async_copy(xflat_hbm.at[src2d.at[0]], rows.at[0], gsem)
        lax.fori_loop(0, RB, cr, 0)
        pltpu.make_async_copy(rows.at[(RB - 1) % 2],
                              tx1_sh.at[dst2d.at[RB - 1]], ssem).wait()

    plsc.subcore_barrier()

    pltpu.sync_copy(tx1_sh.at[pl.ds(ORPT * s, ORPT)],
                    out_tx.at[c, pl.ds(ORPT * s, ORPT)])


@jax.jit
def _spmm_sc(src_p, dst_p, w_p, xflat):
    mesh = plsc.VectorSubcoreMesh(core_axis_name="c", subcore_axis_name="s",
                                  num_cores=NC, num_subcores=NS)
    return pl.kernel(
        _spmm_body,
        out_type=(
            jax.ShapeDtypeStruct((NC, DEG_N, FH), jnp.float32),
            jax.ShapeDtypeStruct((NC, NS, DEG_N), jnp.float32),
            jax.ShapeDtypeStruct((NC, DEG_N), jnp.float32),
        ),
        mesh=mesh,
        compiler_params=pltpu.CompilerParams(needs_layout_passes=False,
                                             use_tc_tiling_on_sc=False),
        scratch_types=[
            pltpu.VMEM((RB, 128), jnp.int32),
            pltpu.VMEM((RB, 128), jnp.int32),
            pltpu.VMEM((RB, 128), jnp.float32),
            pltpu.VMEM((RB, 128), jnp.float32),
            pltpu.VMEM((DEG_N,), jnp.float32),
            pltpu.VMEM((NS, ORPT), jnp.float32),
            pltpu.VMEM((160, FH), jnp.float32),
            pltpu.VMEM((2, 128, FH), jnp.float32),
            pltpu.VMEM_SHARED((DEG_N, FH), jnp.float32),
            pltpu.SemaphoreType.DMA,
            pltpu.SemaphoreType.DMA,
        ],
    )(src_p, dst_p, w_p, xflat)


def _dense_body(x_ref, t0_ref, t1_ref, wz0_ref, wz1a_ref, wz1b_ref,
                wh0_ref, wh1a_ref, wh1b_ref,
                bz_ref, bh_ref, lw_ref, lb_ref, out_ref):
    xb = x_ref[...]
    t0 = t0_ref[0]
    t1 = t1_ref[0]
    z = jax.nn.sigmoid(xb @ wz0_ref[...] + t0 @ wz1a_ref[...]
                       + t1 @ wz1b_ref[...] + bz_ref[...])
    ht = jnp.tanh(xb @ wh0_ref[...] + t0 @ wh1a_ref[...]
                  + t1 @ wh1b_ref[...] + bh_ref[...])
    h = jax.nn.relu((1.0 - z) * ht)
    out_ref[...] = h @ lw_ref[...] + lb_ref[...]


def _dense_block(x, parts, wz0, wz1, wh0, wh1, bz, bh, lw, lb):
    row_spec = pl.BlockSpec((BLK, F), lambda i: (i, 0))
    part0 = pl.BlockSpec((1, BLK, FH), lambda i: (0, i, 0))
    part1 = pl.BlockSpec((1, BLK, FH), lambda i: (1, i, 0))
    full = lambda shape: pl.BlockSpec(shape, lambda i: (0, 0))
    return pl.pallas_call(
        _dense_body,
        grid=(N // BLK,),
        in_specs=[row_spec, part0, part1,
                  full((F, F)), full((FH, F)), full((FH, F)),
                  full((F, F)), full((FH, F)), full((FH, F)),
                  full((1, F)), full((1, F)), full((F, 1)), full((1, 1))],
        out_specs=pl.BlockSpec((BLK, 1), lambda i: (i, 0)),
        out_shape=jax.ShapeDtypeStruct((N, 1), jnp.float32),
    )(x, parts, parts, wz0, wz1[:FH], wz1[FH:], wh0, wh1[:FH], wh1[FH:],
      bz, bh, lw, lb)


def kernel(x, edge_index, edge_weight,
           xz_W0, xz_W1, xz_b, hz_W0, hz_W1, hz_b,
           xr_W0, xr_W1, xr_b, hr_W0, hr_W1, hr_b,
           xh_W0, xh_W1, xh_b, hh_W0, hh_W1, hh_b,
           lin_W, lin_b):
    def pad_blocks(a, pad_val):
        a = a.reshape(NW, EB)
        a = jnp.pad(a, ((0, 0), (0, EBP - EB)), constant_values=pad_val)
        return a.reshape(NW, RB, 128)

    src_p = pad_blocks(edge_index[0], 0)
    dst_p = pad_blocks(edge_index[1], 0)
    w_p = pad_blocks(edge_weight, 0.0)

    xflat = jnp.concatenate([x[:, :FH], x[:, FH:]], axis=0)

    parts, _, _ = _spmm_sc(src_p, dst_p, w_p, xflat)

    bz = (xz_b + hz_b).reshape(1, F)
    bh = (xh_b + hh_b).reshape(1, F)
    return _dense_block(x, parts, xz_W0, xz_W1, xh_W0, xh_W1,
                        bz, bh, lin_W, lin_b.reshape(1, 1))
